# baseline (speedup 1.0000x reference)
import functools

import jax
import jax.numpy as jnp
from jax import lax
from jax.experimental import pallas as pl
from jax.experimental.pallas import tpu as pltpu
from jax.experimental.pallas import tpu_sc as plsc

N = 10000
E = 320000
IN_C = 128
H = 64
OUT_C = 112
L = 28
EPS_GEN = 1e-7
EPS_LN = 1e-5

NC = 2
NS = 16
TILES = NC * NS
EDGES_PER_TILE = E // TILES
CHUNK = 80
NCHUNKS = EDGES_PER_TILE // CHUNK
NPAD = 10240
ROWS_PER_TILE = NPAD // NS
TW = 2 * H


def _sc_edge_body(tab_hbm, src_hbm, dst_hbm, acc_out, acc, sidx_all,
                  didx_all, rows0, rows1, ew, gsem0, gsem1, ssem):
    c = lax.axis_index("c")
    s = lax.axis_index("s")
    w = c * NS + s
    rows = (rows0, rows1)
    gsem = (gsem0, gsem1)

    zero16 = jnp.zeros((16,), jnp.float32)

    def zrow(i, _):
        for j in range(TW // 16):
            ew[i, pl.ds(j * 16, 16)] = zero16
        return 0

    lax.fori_loop(0, CHUNK, zrow, 0)

    rbase = s * ROWS_PER_TILE

    def zcopy(k, _):
        pltpu.async_copy(ew, acc.at[pl.ds(rbase + k * CHUNK, CHUNK)],
                         ssem)
        return 0

    lax.fori_loop(0, ROWS_PER_TILE // CHUNK, zcopy, 0)

    pltpu.sync_copy(src_hbm.at[w], sidx_all)
    pltpu.sync_copy(dst_hbm.at[w], didx_all)

    def zdrain(k, _):
        pltpu.make_async_copy(ew, acc.at[pl.ds(rbase + k * CHUNK, CHUNK)],
                              ssem).wait()
        return 0

    lax.fori_loop(0, ROWS_PER_TILE // CHUNK, zdrain, 0)
    plsc.subcore_barrier()

    def gidx(ci):
        return sidx_all.at[pl.ds(ci * CHUNK, CHUNK)]

    pltpu.async_copy(tab_hbm.at[gidx(0)], rows0, gsem0)

    def half(ci, b):
        nb = 1 - b

        @pl.when(ci < NCHUNKS - 1)
        def _():
            pltpu.async_copy(tab_hbm.at[gidx(ci + 1)], rows[nb],
                             gsem[nb])

        pltpu.make_async_copy(tab_hbm.at[gidx(ci)], rows[b],
                              gsem[b]).wait()

        @pl.when(ci >= 1)
        def _():
            pltpu.make_async_copy(ew, acc.at[didx_all.at[ci - 1]],
                                  ssem).wait()

        @plsc.parallel_loop(0, CHUNK, 1, unroll=16)
        def _(i):
            for j in range(H // 16):
                u = rows[b][i, pl.ds(16 * j, 16)]
                e = jnp.exp(u)
                ew[i, pl.ds(16 * j, 16)] = e
                ew[i, pl.ds(H + 16 * j, 16)] = u * e

        pltpu.async_copy(ew, acc.at[didx_all.at[ci]], ssem, add=True)

    def chunk_body(i, _):
        half(2 * i, 0)
        half(2 * i + 1, 1)
        return 0

    lax.fori_loop(0, NCHUNKS // 2, chunk_body, 0)
    if NCHUNKS % 2:
        half(NCHUNKS - 1, (NCHUNKS - 1) % 2)
    pltpu.make_async_copy(ew, acc.at[didx_all.at[NCHUNKS - 1]],
                          ssem).wait()
    plsc.subcore_barrier()

    pltpu.sync_copy(acc.at[pl.ds(rbase, ROWS_PER_TILE)],
                    acc_out.at[c, pl.ds(rbase, ROWS_PER_TILE)])


_sc_edge = pl.kernel(
    _sc_edge_body,
    out_type=jax.ShapeDtypeStruct((NC, NPAD, TW), jnp.float32),
    mesh=plsc.VectorSubcoreMesh(core_axis_name="c", subcore_axis_name="s"),
    compiler_params=pltpu.CompilerParams(use_tc_tiling_on_sc=False),
    scratch_types=[
        pltpu.VMEM_SHARED((NPAD, TW), jnp.float32),
        pltpu.VMEM((EDGES_PER_TILE,), jnp.int32),
        pltpu.VMEM((NCHUNKS, CHUNK), jnp.int32),
        pltpu.VMEM((CHUNK, H), jnp.float32),
        pltpu.VMEM((CHUNK, H), jnp.float32),
        pltpu.VMEM((CHUNK, TW), jnp.float32),
        pltpu.SemaphoreType.DMA,
        pltpu.SemaphoreType.DMA,
        pltpu.SemaphoreType.DMA,
    ],
)


def _ln(h, g, b):
    mu = jnp.mean(h, axis=-1, keepdims=True)
    var = jnp.mean((h - mu) ** 2, axis=-1, keepdims=True)
    return (h - mu) * jax.lax.rsqrt(var + EPS_LN) * g + b


def _table(h, g, b, t_row):
    return jnp.maximum(_ln(h, g, b), 0.0) * t_row


def _enc_body(x_ref, we_ref, be_ref, g_ref, b_ref, t_ref, h_out, tab_out):
    h = jnp.dot(x_ref[...], we_ref[...],
                preferred_element_type=jnp.float32) + be_ref[...]
    h_out[...] = h
    tab_out[...] = _table(h, g_ref[...], b_ref[...], t_ref[...])


def _layer_body(h_ref, acc_ref, g_ref, b_ref, it_ref, w1m_ref, b1_ref,
                mg_ref, mb_ref, w2m_ref, b2_ref, gn_ref, bn_ref, tn_ref,
                h_out, tab_out):
    h = h_ref[...]
    y = jnp.maximum(_ln(h, g_ref[...], b_ref[...]), 0.0)
    a = acc_ref[0] + acc_ref[1]
    denom = a[:N, :H]
    wsum = a[:N, H:]
    aggr = (wsum * it_ref[...] + EPS_GEN * denom) / (denom + 1e-16)
    out = aggr + y
    z = jnp.dot(out, w1m_ref[...],
                preferred_element_type=jnp.float32) + b1_ref[...]
    z = jnp.maximum(_ln(z, mg_ref[...], mb_ref[...]), 0.0)
    z = jnp.dot(z, w2m_ref[...],
                preferred_element_type=jnp.float32) + b2_ref[...]
    hn = h + z
    h_out[...] = hn
    tab_out[...] = _table(hn, gn_ref[...], bn_ref[...], tn_ref[...])


def _head_body(h_ref, wo_ref, bo_ref, o_ref):
    o_ref[...] = jnp.dot(h_ref[...], wo_ref[...],
                         preferred_element_type=jnp.float32) + bo_ref[...]


_f32 = jnp.float32


def _tc_call(body, out_shapes):
    return pl.pallas_call(
        body,
        out_shape=tuple(jax.ShapeDtypeStruct(s, _f32) for s in out_shapes),
    )


_enc = _tc_call(_enc_body, ((N, H), (N, H)))
_layer = _tc_call(_layer_body, ((N, H), (N, H)))
_head = _tc_call(_head_body, ((N, OUT_C),))


def kernel(x, edge_index, W_enc, b_enc, ln_g, ln_b, t, W1, b1, mg, mb,
           W2, b2, W_out, b_out):
    src = jnp.reshape(edge_index[0], (TILES, EDGES_PER_TILE))
    dst = jnp.reshape(edge_index[1], (TILES, NCHUNKS, CHUNK))
    row = lambda v: jnp.reshape(v, (1, -1))
    t_rows = jnp.broadcast_to(t[:, None], (L, H))
    it_rows = jnp.broadcast_to((1.0 / t)[:, None], (L, H))

    h, tab = _enc(x, W_enc, row(b_enc), row(ln_g[0]), row(ln_b[0]),
                  row(t_rows[0]))
    for l in range(L):
        acc = _sc_edge(tab, src, dst)
        nl = (l + 1) % L
        h, tab = _layer(h, acc,
                        row(ln_g[l]), row(ln_b[l]), row(it_rows[l]),
                        W1[l], row(b1[l]), row(mg[l]), row(mb[l]),
                        W2[l], row(b2[l]),
                        row(ln_g[nl]), row(ln_b[nl]), row(t_rows[nl]))
    (out,) = _head(h, W_out, row(b_out))
    return out


# speedup vs baseline: 1.0464x; 1.0464x over previous
import functools

import jax
import jax.numpy as jnp
from jax import lax
from jax.experimental import pallas as pl
from jax.experimental.pallas import tpu as pltpu
from jax.experimental.pallas import tpu_sc as plsc

N = 10000
E = 320000
IN_C = 128
H = 64
OUT_C = 112
L = 28
EPS_GEN = 1e-7
EPS_LN = 1e-5

NC = 2
NS = 16
TILES = NC * NS
EDGES_PER_TILE = E // TILES
CHUNK = 80
NCHUNKS = EDGES_PER_TILE // CHUNK
NPAD = 10240
ROWS_PER_TILE = NPAD // NS
TW = 2 * H


def _sc_edge_body(tab_hbm, src_hbm, dst_hbm, acc_out, acc, sidx_all,
                  didx_all, rows0, rows1, ew, gsem0, gsem1, ssem):
    c = lax.axis_index("c")
    s = lax.axis_index("s")
    w = c * NS + s
    rows = (rows0, rows1)
    gsem = (gsem0, gsem1)

    zero16 = jnp.zeros((16,), jnp.float32)

    def zrow(i, _):
        for j in range(TW // 16):
            ew[i, pl.ds(j * 16, 16)] = zero16
        return 0

    lax.fori_loop(0, CHUNK, zrow, 0)

    rbase = s * ROWS_PER_TILE

    def zcopy(k, _):
        pltpu.async_copy(ew, acc.at[pl.ds(rbase + k * CHUNK, CHUNK)],
                         ssem)
        return 0

    lax.fori_loop(0, ROWS_PER_TILE // CHUNK, zcopy, 0)

    pltpu.sync_copy(src_hbm.at[w], sidx_all)
    pltpu.sync_copy(dst_hbm.at[w], didx_all)

    def zdrain(k, _):
        pltpu.make_async_copy(ew, acc.at[pl.ds(rbase + k * CHUNK, CHUNK)],
                              ssem).wait()
        return 0

    lax.fori_loop(0, ROWS_PER_TILE // CHUNK, zdrain, 0)
    plsc.subcore_barrier()

    def gidx(ci):
        return sidx_all.at[pl.ds(ci * CHUNK, CHUNK)]

    pltpu.async_copy(tab_hbm.at[gidx(0)], rows0, gsem0)

    def half(ci, b):
        nb = 1 - b

        @pl.when(ci < NCHUNKS - 1)
        def _():
            pltpu.async_copy(tab_hbm.at[gidx(ci + 1)], rows[nb],
                             gsem[nb])

        pltpu.make_async_copy(tab_hbm.at[gidx(ci)], rows[b],
                              gsem[b]).wait()

        @pl.when(ci >= 1)
        def _():
            pltpu.make_async_copy(ew, acc.at[didx_all.at[ci - 1]],
                                  ssem).wait()

        @plsc.parallel_loop(0, CHUNK, 1, unroll=8)
        def _(i):
            for j in range(H // 16):
                u = rows[b][i, pl.ds(16 * j, 16)]
                e = jnp.exp(u)
                ew[i, pl.ds(16 * j, 16)] = e
                ew[i, pl.ds(H + 16 * j, 16)] = u * e

        pltpu.async_copy(ew, acc.at[didx_all.at[ci]], ssem, add=True)

    def chunk_body(i, _):
        half(2 * i, 0)
        half(2 * i + 1, 1)
        return 0

    lax.fori_loop(0, NCHUNKS // 2, chunk_body, 0)
    if NCHUNKS % 2:
        half(NCHUNKS - 1, (NCHUNKS - 1) % 2)
    pltpu.make_async_copy(ew, acc.at[didx_all.at[NCHUNKS - 1]],
                          ssem).wait()
    plsc.subcore_barrier()

    pltpu.sync_copy(acc.at[pl.ds(rbase, ROWS_PER_TILE)],
                    acc_out.at[c, pl.ds(rbase, ROWS_PER_TILE)])


_sc_edge = pl.kernel(
    _sc_edge_body,
    out_type=jax.ShapeDtypeStruct((NC, NPAD, TW), jnp.float32),
    mesh=plsc.VectorSubcoreMesh(core_axis_name="c", subcore_axis_name="s"),
    compiler_params=pltpu.CompilerParams(use_tc_tiling_on_sc=False),
    scratch_types=[
        pltpu.VMEM_SHARED((NPAD, TW), jnp.float32),
        pltpu.VMEM((EDGES_PER_TILE,), jnp.int32),
        pltpu.VMEM((NCHUNKS, CHUNK), jnp.int32),
        pltpu.VMEM((CHUNK, H), jnp.float32),
        pltpu.VMEM((CHUNK, H), jnp.float32),
        pltpu.VMEM((CHUNK, TW), jnp.float32),
        pltpu.SemaphoreType.DMA,
        pltpu.SemaphoreType.DMA,
        pltpu.SemaphoreType.DMA,
    ],
)


def _ln(h, g, b):
    mu = jnp.mean(h, axis=-1, keepdims=True)
    var = jnp.mean((h - mu) ** 2, axis=-1, keepdims=True)
    return (h - mu) * jax.lax.rsqrt(var + EPS_LN) * g + b


def _table(h, g, b, t_row):
    return jnp.maximum(_ln(h, g, b), 0.0) * t_row


def _enc_body(x_ref, we_ref, be_ref, g_ref, b_ref, t_ref, h_out, tab_out):
    h = jnp.dot(x_ref[...], we_ref[...],
                preferred_element_type=jnp.float32) + be_ref[...]
    h_out[...] = h
    tab_out[...] = _table(h, g_ref[...], b_ref[...], t_ref[...])


def _layer_body(h_ref, acc_ref, g_ref, b_ref, it_ref, w1m_ref, b1_ref,
                mg_ref, mb_ref, w2m_ref, b2_ref, gn_ref, bn_ref, tn_ref,
                h_out, tab_out):
    h = h_ref[...]
    y = jnp.maximum(_ln(h, g_ref[...], b_ref[...]), 0.0)
    a = acc_ref[0] + acc_ref[1]
    denom = a[:N, :H]
    wsum = a[:N, H:]
    aggr = (wsum * it_ref[...] + EPS_GEN * denom) / (denom + 1e-16)
    out = aggr + y
    z = jnp.dot(out, w1m_ref[...],
                preferred_element_type=jnp.float32) + b1_ref[...]
    z = jnp.maximum(_ln(z, mg_ref[...], mb_ref[...]), 0.0)
    z = jnp.dot(z, w2m_ref[...],
                preferred_element_type=jnp.float32) + b2_ref[...]
    hn = h + z
    h_out[...] = hn
    tab_out[...] = _table(hn, gn_ref[...], bn_ref[...], tn_ref[...])


def _head_body(h_ref, wo_ref, bo_ref, o_ref):
    o_ref[...] = jnp.dot(h_ref[...], wo_ref[...],
                         preferred_element_type=jnp.float32) + bo_ref[...]


_f32 = jnp.float32


def _tc_call(body, out_shapes):
    return pl.pallas_call(
        body,
        out_shape=tuple(jax.ShapeDtypeStruct(s, _f32) for s in out_shapes),
    )


_enc = _tc_call(_enc_body, ((N, H), (N, H)))
_layer = _tc_call(_layer_body, ((N, H), (N, H)))
_head = _tc_call(_head_body, ((N, OUT_C),))


def kernel(x, edge_index, W_enc, b_enc, ln_g, ln_b, t, W1, b1, mg, mb,
           W2, b2, W_out, b_out):
    src = jnp.reshape(edge_index[0], (TILES, EDGES_PER_TILE))
    dst = jnp.reshape(edge_index[1], (TILES, NCHUNKS, CHUNK))
    row = lambda v: jnp.reshape(v, (1, -1))
    t_rows = jnp.broadcast_to(t[:, None], (L, H))
    it_rows = jnp.broadcast_to((1.0 / t)[:, None], (L, H))

    h, tab = _enc(x, W_enc, row(b_enc), row(ln_g[0]), row(ln_b[0]),
                  row(t_rows[0]))
    for l in range(L):
        acc = _sc_edge(tab, src, dst)
        nl = (l + 1) % L
        h, tab = _layer(h, acc,
                        row(ln_g[l]), row(ln_b[l]), row(it_rows[l]),
                        W1[l], row(b1[l]), row(mg[l]), row(mb[l]),
                        W2[l], row(b2[l]),
                        row(ln_g[nl]), row(ln_b[nl]), row(t_rows[nl]))
    (out,) = _head(h, W_out, row(b_out))
    return out
